# baseline (speedup 1.0000x reference)
import jax
import jax.numpy as jnp
from jax.experimental import pallas as pl
from jax.experimental.pallas import tpu as pltpu

_AT = 0.6
_BT = 0.4
_ST = 0.8
_TOPK_RAT = 4
_W = 0.5
_PREC = jax.lax.Precision.DEFAULT
_TJ = 512


def _mm(a, b):
    return jax.lax.dot_general(a, b, (((1,), (0,)), ((), ())),
                               precision=_PREC,
                               preferred_element_type=jnp.float32)


def _mmb(a, b):
    return jax.lax.dot_general(a.astype(jnp.bfloat16), b.astype(jnp.bfloat16),
                               (((1,), (0,)), ((), ())),
                               precision=_PREC,
                               preferred_element_type=jnp.float32)


def _lrelu(x):
    return jnp.where(x >= 0, x, 0.2 * x)


def _shift_right(y, col):
    r = pltpu.roll(y, 1, 1)
    return jnp.where(col >= 1, r, 0.0)


def _shift_left(y, col, T):
    r = pltpu.roll(y, T - 1, 1)
    return jnp.where(col <= T - 2, r, 0.0)


def _attention_body(x_ref, w1_ref, b1_ref, w2_ref, b2_ref, w3_ref, b3_ref,
                    o_ref):
    x = x_ref[0, 0]
    T = x.shape[1]
    w1 = w1_ref[0]
    w2 = w2_ref[0]
    col512 = jax.lax.broadcasted_iota(jnp.int32, (w1.shape[1], T), 1)
    h = _mm(w1[1], x)
    h = h + _shift_right(_mm(w1[0], x), col512)
    h = h + _shift_left(_mm(w1[2], x), col512, T)
    h = _lrelu(h + b1_ref[0])
    g = _mm(w2[1], h)
    g = g + _shift_right(_mm(w2[0], h), col512)
    g = g + _shift_left(_mm(w2[2], h), col512, T)
    g = _lrelu(g + b2_ref[0])
    a = _mm(w3_ref[0], g) + b3_ref[0]
    o_ref[0, 0] = jax.nn.sigmoid(a)


def _adj_body(feat_ref, va_ref, fa_ref, vat_ref, fat_ref,
              aab_ref, amb_ref, avg_ref):
    s = pl.program_id(0)
    j = pl.program_id(2)
    feat = feat_ref[0, 0]
    T = feat.shape[1]
    TJ = aab_ref.shape[-1]
    j0 = j * TJ
    k = T // _TOPK_RAT
    vat = vat_ref[0]
    fat = fat_ref[0]
    va_t = va_ref[0, :, pl.ds(j0, TJ)]
    fa_t = fa_ref[0, :, pl.ds(j0, TJ)]

    armT = ((vat >= _AT) & (fat >= _AT)).astype(jnp.float32)
    brmT = ((vat < _BT) & (fat < _BT)).astype(jnp.float32)
    abrT = armT + brmT
    arm_c = ((va_t >= _AT) & (fa_t >= _AT)).astype(jnp.float32)
    brm_c = ((va_t < _BT) & (fa_t < _BT)).astype(jnp.float32)
    abr_c = arm_c + brm_c
    ambr_c = 1.0 - abr_c

    onesD = jnp.ones((1, feat.shape[0]), jnp.float32)
    nrm = jnp.sqrt(_mm(onesD, feat * feat))
    fn = feat / jnp.maximum(nrm, 1e-12)
    featc = feat_ref[0, 0, :, pl.ds(j0, TJ)]
    nrm_c = jnp.sqrt(_mm(onesD, featc * featc))
    fnc = featc / jnp.maximum(nrm_c, 1e-12)
    S = jax.lax.dot_general(fn, fnc, (((0,), (0,)), ((), ())),
                            precision=_PREC,
                            preferred_element_type=jnp.float32)
    S = jnp.where(S < _ST, 0.0, S)

    onesT = jnp.ones((1, T), jnp.bfloat16)

    def bis(_, carry):
        lo, hi = carry
        mid = 0.5 * (lo + hi)
        cnt = _mmb(onesT, (S >= mid).astype(jnp.bfloat16))
        ge = cnt >= k
        return jnp.where(ge, mid, lo), jnp.where(ge, hi, mid)

    c8 = _mmb(onesT, (S > 0).astype(jnp.bfloat16))
    many = c8 >= k
    lo0 = jnp.where(many, jnp.float32(_ST), 0.0)
    hi0 = jnp.where(many, jnp.float32(1.01), jnp.float32(_ST))
    lo, _ = jax.lax.fori_loop(0, 24, bis, (lo0, hi0))
    S = jnp.where(S >= lo, S, 0.0)

    ii = jax.lax.broadcasted_iota(jnp.int32, (T, TJ), 0)
    jj = jax.lax.broadcasted_iota(jnp.int32, (T, TJ), 1) + j0
    iseye = ii == jj
    eye_m = iseye & (abr_c != 1.0)
    pos = (abrT * ambr_c) > 0
    keep_v = pos | eye_m
    keep_f = (~pos) | eye_m

    onesTf = jnp.ones((1, T), jnp.float32)

    def l1n(A):
        cs = _mm(onesTf, A)
        return A / jnp.maximum(cs, 1e-12)

    A_act = l1n(S * arm_c * armT)
    A_bg = l1n(S * brm_c * brmT)
    A_amb = l1n(jnp.where(s == 0, jnp.where(keep_v, S, 0.0),
                          jnp.where(keep_f, S, 0.0)))

    aab_ref[0, 0, 0] = A_act.astype(jnp.bfloat16)
    aab_ref[0, 0, 1] = A_bg.astype(jnp.bfloat16)
    amb_ref[0, 0] = A_amb.astype(jnp.bfloat16)

    diagv = jnp.sum(jnp.where(iseye, A_amb, 0.0), axis=0, keepdims=True)
    avg = _mmb(feat, A_act + A_bg + A_amb) + featc * diagv
    avg_ref[0, 0] = avg.astype(jnp.bfloat16)


def _gcn_body(feat_ref, w1_ref, b1_ref, w2_ref, b2_ref, a_ref, amb_ref,
              vat_ref, fat_ref, p_ref):
    r = pl.program_id(1)
    feat = feat_ref[0, 0]
    A = a_ref[0, 0, 0]
    x = _lrelu(_mmb(_mmb(w1_ref[0, 0], feat) + b1_ref[0, 0], A))
    x = _lrelu(_mmb(_mmb(w2_ref[0, 0], x) + b2_ref[0, 0], A))
    Am = amb_ref[0, 0]
    vat = vat_ref[0]
    fat = fat_ref[0]
    armT = ((vat >= _AT) & (fat >= _AT)).astype(jnp.float32)
    brmT = ((vat < _BT) & (fat < _BT)).astype(jnp.float32)
    ma = jnp.where(jnp.sum(armT) > 0, armT, 0.0)
    mb = jnp.where(jnp.sum(brmT) > 0, brmT, 0.0)
    mask = jnp.where(r == 0, ma, mb).astype(jnp.bfloat16)
    y = _mmb(x, Am * mask)
    p_ref[0, 0, 0] = (x + y).astype(jnp.bfloat16)


def _blend_body(feat_ref, avg_ref, p_ref, out_ref):
    feat = feat_ref[0, 0]
    tot = (avg_ref[0, 0].astype(jnp.float32)
           + p_ref[0, 0, 0].astype(jnp.float32)
           + p_ref[0, 0, 1].astype(jnp.float32))
    out_ref[0, 0] = _W * feat + (1.0 - _W) * 0.5 * tot


def _attention(xs, w1s, b1s, w2s, b2s, w3s, b3s):
    S2, B, D, T = xs.shape
    C = w1s.shape[2]
    return pl.pallas_call(
        _attention_body,
        grid=(S2, B),
        in_specs=[
            pl.BlockSpec((1, 1, D, T), lambda s, b: (s, b, 0, 0)),
            pl.BlockSpec((1, 3, C, D), lambda s, b: (s, 0, 0, 0)),
            pl.BlockSpec((1, C, 1), lambda s, b: (s, 0, 0)),
            pl.BlockSpec((1, 3, C, C), lambda s, b: (s, 0, 0, 0)),
            pl.BlockSpec((1, C, 1), lambda s, b: (s, 0, 0)),
            pl.BlockSpec((1, 1, C), lambda s, b: (s, 0, 0)),
            pl.BlockSpec((1, 1, 1), lambda s, b: (s, 0, 0)),
        ],
        out_specs=pl.BlockSpec((1, 1, 1, T), lambda s, b: (s, b, 0, 0)),
        out_shape=jax.ShapeDtypeStruct((S2, B, 1, T), jnp.float32),
    )(xs, w1s, b1s, w2s, b2s, w3s, b3s)


def kernel(vfeat, ffeat, avW1, avb1, avW2, avb2, avW3, avb3,
           afW1, afb1, afW2, afb2, afW3, afb3,
           agvW1, agvb1, agvW2, agvb2, bgvW1, bgvb1, bgvW2, bgvb2,
           agfW1, agfb1, agfW2, agfb2, bgfW1, bgfb1, bgfW2, bgfb2,
           is_training):
    B, D, T = vfeat.shape
    TJ = _TJ
    J = T // TJ
    xs = jnp.stack([vfeat, ffeat])
    w1s = jnp.stack([jnp.transpose(avW1, (2, 0, 1)),
                     jnp.transpose(afW1, (2, 0, 1))])
    b1s = jnp.stack([avb1, afb1])[:, :, None]
    w2s = jnp.stack([jnp.transpose(avW2, (2, 0, 1)),
                     jnp.transpose(afW2, (2, 0, 1))])
    b2s = jnp.stack([avb2, afb2])[:, :, None]
    w3s = jnp.stack([avW3[:, :, 0], afW3[:, :, 0]])
    b3s = jnp.stack([avb3, afb3])[:, :, None]

    atn = _attention(xs, w1s, b1s, w2s, b2s, w3s, b3s)
    vatn, fatn = atn[0], atn[1]
    vatn_t = jnp.transpose(vatn, (0, 2, 1))
    fatn_t = jnp.transpose(fatn, (0, 2, 1))

    aab, aamb, avgd = pl.pallas_call(
        _adj_body,
        grid=(2, B, J),
        in_specs=[
            pl.BlockSpec((1, 1, D, T), lambda s, b, j: (s, b, 0, 0)),
            pl.BlockSpec((1, 1, T), lambda s, b, j: (b, 0, 0)),
            pl.BlockSpec((1, 1, T), lambda s, b, j: (b, 0, 0)),
            pl.BlockSpec((1, T, 1), lambda s, b, j: (b, 0, 0)),
            pl.BlockSpec((1, T, 1), lambda s, b, j: (b, 0, 0)),
        ],
        out_specs=[
            pl.BlockSpec((1, 1, 2, T, TJ), lambda s, b, j: (s, b, 0, 0, j)),
            pl.BlockSpec((1, 1, T, TJ), lambda s, b, j: (s, b, 0, j)),
            pl.BlockSpec((1, 1, D, TJ), lambda s, b, j: (s, b, 0, j)),
        ],
        out_shape=[
            jax.ShapeDtypeStruct((2, B, 2, T, T), jnp.bfloat16),
            jax.ShapeDtypeStruct((2, B, T, T), jnp.bfloat16),
            jax.ShapeDtypeStruct((2, B, D, T), jnp.bfloat16),
        ],
    )(xs, vatn, fatn, vatn_t, fatn_t)

    wg1 = jnp.stack([jnp.stack([agvW1, bgvW1]),
                     jnp.stack([agfW1, bgfW1])]).astype(jnp.bfloat16)
    bg1 = jnp.stack([jnp.stack([agvb1, bgvb1]),
                     jnp.stack([agfb1, bgfb1])])[..., None]
    wg2 = jnp.stack([jnp.stack([agvW2, bgvW2]),
                     jnp.stack([agfW2, bgfW2])]).astype(jnp.bfloat16)
    bg2 = jnp.stack([jnp.stack([agvb2, bgvb2]),
                     jnp.stack([agfb2, bgfb2])])[..., None]

    xs_bf = xs.astype(jnp.bfloat16)
    pab = pl.pallas_call(
        _gcn_body,
        grid=(2, 2, B),
        in_specs=[
            pl.BlockSpec((1, 1, D, T), lambda s, r, b: (s, b, 0, 0)),
            pl.BlockSpec((1, 1, D, D), lambda s, r, b: (s, r, 0, 0)),
            pl.BlockSpec((1, 1, D, 1), lambda s, r, b: (s, r, 0, 0)),
            pl.BlockSpec((1, 1, D, D), lambda s, r, b: (s, r, 0, 0)),
            pl.BlockSpec((1, 1, D, 1), lambda s, r, b: (s, r, 0, 0)),
            pl.BlockSpec((1, 1, 1, T, T), lambda s, r, b: (s, b, r, 0, 0)),
            pl.BlockSpec((1, 1, T, T), lambda s, r, b: (s, b, 0, 0)),
            pl.BlockSpec((1, T, 1), lambda s, r, b: (b, 0, 0)),
            pl.BlockSpec((1, T, 1), lambda s, r, b: (b, 0, 0)),
        ],
        out_specs=pl.BlockSpec((1, 1, 1, D, T),
                               lambda s, r, b: (s, b, r, 0, 0)),
        out_shape=jax.ShapeDtypeStruct((2, B, 2, D, T), jnp.bfloat16),
    )(xs_bf, wg1, bg1, wg2, bg2, aab, aamb, vatn_t, fatn_t)

    new = pl.pallas_call(
        _blend_body,
        grid=(2, B, J),
        in_specs=[
            pl.BlockSpec((1, 1, D, TJ), lambda s, b, j: (s, b, 0, j)),
            pl.BlockSpec((1, 1, D, TJ), lambda s, b, j: (s, b, 0, j)),
            pl.BlockSpec((1, 1, 2, D, TJ), lambda s, b, j: (s, b, 0, 0, j)),
        ],
        out_specs=pl.BlockSpec((1, 1, D, TJ), lambda s, b, j: (s, b, 0, j)),
        out_shape=jax.ShapeDtypeStruct((2, B, D, T), jnp.float32),
    )(xs, avgd, pab)

    atn2 = _attention(new, w1s, b1s, w2s, b2s, w3s, b3s)
    return atn2[0], new[0], atn2[1], new[1]


# speedup vs baseline: 1.0268x; 1.0268x over previous
import jax
import jax.numpy as jnp
from jax.experimental import pallas as pl
from jax.experimental.pallas import tpu as pltpu

_AT = 0.6
_BT = 0.4
_ST = 0.8
_TOPK_RAT = 4
_W = 0.5
_PREC = jax.lax.Precision.DEFAULT
_TJ = 512


def _mm(a, b):
    return jax.lax.dot_general(a, b, (((1,), (0,)), ((), ())),
                               precision=_PREC,
                               preferred_element_type=jnp.float32)


def _mmb(a, b):
    return jax.lax.dot_general(a.astype(jnp.bfloat16), b.astype(jnp.bfloat16),
                               (((1,), (0,)), ((), ())),
                               precision=_PREC,
                               preferred_element_type=jnp.float32)


def _lrelu(x):
    return jnp.where(x >= 0, x, 0.2 * x)


def _shift_right(y, col):
    r = pltpu.roll(y, 1, 1)
    return jnp.where(col >= 1, r, 0.0)


def _shift_left(y, col, T):
    r = pltpu.roll(y, T - 1, 1)
    return jnp.where(col <= T - 2, r, 0.0)


def _attention_body(x_ref, w1_ref, b1_ref, w2_ref, b2_ref, w3_ref, b3_ref,
                    o_ref):
    x = x_ref[0, 0]
    T = x.shape[1]
    w1 = w1_ref[0]
    w2 = w2_ref[0]
    col512 = jax.lax.broadcasted_iota(jnp.int32, (w1.shape[1], T), 1)
    h = _mm(w1[1], x)
    h = h + _shift_right(_mm(w1[0], x), col512)
    h = h + _shift_left(_mm(w1[2], x), col512, T)
    h = _lrelu(h + b1_ref[0])
    g = _mm(w2[1], h)
    g = g + _shift_right(_mm(w2[0], h), col512)
    g = g + _shift_left(_mm(w2[2], h), col512, T)
    g = _lrelu(g + b2_ref[0])
    a = _mm(w3_ref[0], g) + b3_ref[0]
    o_ref[0, 0] = jax.nn.sigmoid(a)


def _adj_body(feat_ref, va_ref, fa_ref, vat_ref, fat_ref,
              aab_ref, amb_ref, avg_ref):
    s = pl.program_id(0)
    j = pl.program_id(2)
    feat = feat_ref[0, 0]
    T = feat.shape[1]
    TJ = aab_ref.shape[-1]
    j0 = j * TJ
    k = T // _TOPK_RAT
    vat = vat_ref[0]
    fat = fat_ref[0]
    va_t = va_ref[0, :, pl.ds(j0, TJ)]
    fa_t = fa_ref[0, :, pl.ds(j0, TJ)]

    armT = ((vat >= _AT) & (fat >= _AT)).astype(jnp.float32)
    brmT = ((vat < _BT) & (fat < _BT)).astype(jnp.float32)
    abrT = armT + brmT
    arm_c = ((va_t >= _AT) & (fa_t >= _AT)).astype(jnp.float32)
    brm_c = ((va_t < _BT) & (fa_t < _BT)).astype(jnp.float32)
    abr_c = arm_c + brm_c
    ambr_c = 1.0 - abr_c

    onesD = jnp.ones((1, feat.shape[0]), jnp.float32)
    nrm = jnp.sqrt(_mm(onesD, feat * feat))
    fn = feat / jnp.maximum(nrm, 1e-12)
    featc = feat_ref[0, 0, :, pl.ds(j0, TJ)]
    nrm_c = jnp.sqrt(_mm(onesD, featc * featc))
    fnc = featc / jnp.maximum(nrm_c, 1e-12)
    S = jax.lax.dot_general(fn, fnc, (((0,), (0,)), ((), ())),
                            precision=_PREC,
                            preferred_element_type=jnp.float32)
    S = jnp.where(S < _ST, 0.0, S)

    def bis(_, carry):
        lo, hi = carry
        mid = 0.5 * (lo + hi)
        cnt = jnp.sum((S >= mid).astype(jnp.float32), axis=0, keepdims=True)
        ge = cnt >= k
        return jnp.where(ge, mid, lo), jnp.where(ge, hi, mid)

    c8 = jnp.sum((S > 0).astype(jnp.float32), axis=0, keepdims=True)
    many = c8 >= k
    lo0 = jnp.where(many, jnp.float32(_ST), 0.0)
    hi0 = jnp.where(many, jnp.float32(1.01), jnp.float32(_ST))
    lo, _ = jax.lax.fori_loop(0, 24, bis, (lo0, hi0))
    S = jnp.where(S >= lo, S, 0.0)

    ii = jax.lax.broadcasted_iota(jnp.int32, (T, TJ), 0)
    jj = jax.lax.broadcasted_iota(jnp.int32, (T, TJ), 1) + j0
    iseye = ii == jj
    eye_m = iseye & (abr_c != 1.0)
    pos = (abrT * ambr_c) > 0
    keep_v = pos | eye_m
    keep_f = (~pos) | eye_m

    onesTf = jnp.ones((1, T), jnp.float32)

    def l1n(A):
        cs = _mm(onesTf, A)
        return A / jnp.maximum(cs, 1e-12)

    A_act = l1n(S * arm_c * armT)
    A_bg = l1n(S * brm_c * brmT)
    A_amb = l1n(jnp.where(s == 0, jnp.where(keep_v, S, 0.0),
                          jnp.where(keep_f, S, 0.0)))

    aab_ref[0, 0, 0] = A_act.astype(jnp.bfloat16)
    aab_ref[0, 0, 1] = A_bg.astype(jnp.bfloat16)
    amb_ref[0, 0] = A_amb.astype(jnp.bfloat16)

    diagv = jnp.sum(jnp.where(iseye, A_amb, 0.0), axis=0, keepdims=True)
    avg = _mmb(feat, A_act + A_bg + A_amb) + featc * diagv
    avg_ref[0, 0] = avg.astype(jnp.bfloat16)


def _gcn_body(feat_ref, w1_ref, b1_ref, w2_ref, b2_ref, a_ref, amb_ref,
              vat_ref, fat_ref, p_ref):
    r = pl.program_id(1)
    feat = feat_ref[0, 0]
    A = a_ref[0, 0, 0]
    x = _lrelu(_mmb(_mmb(w1_ref[0, 0], feat) + b1_ref[0, 0], A))
    x = _lrelu(_mmb(_mmb(w2_ref[0, 0], x) + b2_ref[0, 0], A))
    Am = amb_ref[0, 0]
    vat = vat_ref[0]
    fat = fat_ref[0]
    armT = ((vat >= _AT) & (fat >= _AT)).astype(jnp.float32)
    brmT = ((vat < _BT) & (fat < _BT)).astype(jnp.float32)
    ma = jnp.where(jnp.sum(armT) > 0, armT, 0.0)
    mb = jnp.where(jnp.sum(brmT) > 0, brmT, 0.0)
    mask = jnp.where(r == 0, ma, mb).astype(jnp.bfloat16)
    y = _mmb(x, Am * mask)
    p_ref[0, 0, 0] = (x + y).astype(jnp.bfloat16)


def _blend_body(feat_ref, avg_ref, p_ref, out_ref):
    feat = feat_ref[0, 0]
    tot = (avg_ref[0, 0].astype(jnp.float32)
           + p_ref[0, 0, 0].astype(jnp.float32)
           + p_ref[0, 0, 1].astype(jnp.float32))
    out_ref[0, 0] = _W * feat + (1.0 - _W) * 0.5 * tot


def _attention(xs, w1s, b1s, w2s, b2s, w3s, b3s):
    S2, B, D, T = xs.shape
    C = w1s.shape[2]
    return pl.pallas_call(
        _attention_body,
        grid=(S2, B),
        in_specs=[
            pl.BlockSpec((1, 1, D, T), lambda s, b: (s, b, 0, 0)),
            pl.BlockSpec((1, 3, C, D), lambda s, b: (s, 0, 0, 0)),
            pl.BlockSpec((1, C, 1), lambda s, b: (s, 0, 0)),
            pl.BlockSpec((1, 3, C, C), lambda s, b: (s, 0, 0, 0)),
            pl.BlockSpec((1, C, 1), lambda s, b: (s, 0, 0)),
            pl.BlockSpec((1, 1, C), lambda s, b: (s, 0, 0)),
            pl.BlockSpec((1, 1, 1), lambda s, b: (s, 0, 0)),
        ],
        out_specs=pl.BlockSpec((1, 1, 1, T), lambda s, b: (s, b, 0, 0)),
        out_shape=jax.ShapeDtypeStruct((S2, B, 1, T), jnp.float32),
    )(xs, w1s, b1s, w2s, b2s, w3s, b3s)


def kernel(vfeat, ffeat, avW1, avb1, avW2, avb2, avW3, avb3,
           afW1, afb1, afW2, afb2, afW3, afb3,
           agvW1, agvb1, agvW2, agvb2, bgvW1, bgvb1, bgvW2, bgvb2,
           agfW1, agfb1, agfW2, agfb2, bgfW1, bgfb1, bgfW2, bgfb2,
           is_training):
    B, D, T = vfeat.shape
    TJ = _TJ
    J = T // TJ
    xs = jnp.stack([vfeat, ffeat])
    w1s = jnp.stack([jnp.transpose(avW1, (2, 0, 1)),
                     jnp.transpose(afW1, (2, 0, 1))])
    b1s = jnp.stack([avb1, afb1])[:, :, None]
    w2s = jnp.stack([jnp.transpose(avW2, (2, 0, 1)),
                     jnp.transpose(afW2, (2, 0, 1))])
    b2s = jnp.stack([avb2, afb2])[:, :, None]
    w3s = jnp.stack([avW3[:, :, 0], afW3[:, :, 0]])
    b3s = jnp.stack([avb3, afb3])[:, :, None]

    atn = _attention(xs, w1s, b1s, w2s, b2s, w3s, b3s)
    vatn, fatn = atn[0], atn[1]
    vatn_t = jnp.transpose(vatn, (0, 2, 1))
    fatn_t = jnp.transpose(fatn, (0, 2, 1))

    aab, aamb, avgd = pl.pallas_call(
        _adj_body,
        grid=(2, B, J),
        in_specs=[
            pl.BlockSpec((1, 1, D, T), lambda s, b, j: (s, b, 0, 0)),
            pl.BlockSpec((1, 1, T), lambda s, b, j: (b, 0, 0)),
            pl.BlockSpec((1, 1, T), lambda s, b, j: (b, 0, 0)),
            pl.BlockSpec((1, T, 1), lambda s, b, j: (b, 0, 0)),
            pl.BlockSpec((1, T, 1), lambda s, b, j: (b, 0, 0)),
        ],
        out_specs=[
            pl.BlockSpec((1, 1, 2, T, TJ), lambda s, b, j: (s, b, 0, 0, j)),
            pl.BlockSpec((1, 1, T, TJ), lambda s, b, j: (s, b, 0, j)),
            pl.BlockSpec((1, 1, D, TJ), lambda s, b, j: (s, b, 0, j)),
        ],
        out_shape=[
            jax.ShapeDtypeStruct((2, B, 2, T, T), jnp.bfloat16),
            jax.ShapeDtypeStruct((2, B, T, T), jnp.bfloat16),
            jax.ShapeDtypeStruct((2, B, D, T), jnp.bfloat16),
        ],
    )(xs, vatn, fatn, vatn_t, fatn_t)

    wg1 = jnp.stack([jnp.stack([agvW1, bgvW1]),
                     jnp.stack([agfW1, bgfW1])]).astype(jnp.bfloat16)
    bg1 = jnp.stack([jnp.stack([agvb1, bgvb1]),
                     jnp.stack([agfb1, bgfb1])])[..., None]
    wg2 = jnp.stack([jnp.stack([agvW2, bgvW2]),
                     jnp.stack([agfW2, bgfW2])]).astype(jnp.bfloat16)
    bg2 = jnp.stack([jnp.stack([agvb2, bgvb2]),
                     jnp.stack([agfb2, bgfb2])])[..., None]

    xs_bf = xs.astype(jnp.bfloat16)
    pab = pl.pallas_call(
        _gcn_body,
        grid=(2, 2, B),
        in_specs=[
            pl.BlockSpec((1, 1, D, T), lambda s, r, b: (s, b, 0, 0)),
            pl.BlockSpec((1, 1, D, D), lambda s, r, b: (s, r, 0, 0)),
            pl.BlockSpec((1, 1, D, 1), lambda s, r, b: (s, r, 0, 0)),
            pl.BlockSpec((1, 1, D, D), lambda s, r, b: (s, r, 0, 0)),
            pl.BlockSpec((1, 1, D, 1), lambda s, r, b: (s, r, 0, 0)),
            pl.BlockSpec((1, 1, 1, T, T), lambda s, r, b: (s, b, r, 0, 0)),
            pl.BlockSpec((1, 1, T, T), lambda s, r, b: (s, b, 0, 0)),
            pl.BlockSpec((1, T, 1), lambda s, r, b: (b, 0, 0)),
            pl.BlockSpec((1, T, 1), lambda s, r, b: (b, 0, 0)),
        ],
        out_specs=pl.BlockSpec((1, 1, 1, D, T),
                               lambda s, r, b: (s, b, r, 0, 0)),
        out_shape=jax.ShapeDtypeStruct((2, B, 2, D, T), jnp.bfloat16),
    )(xs_bf, wg1, bg1, wg2, bg2, aab, aamb, vatn_t, fatn_t)

    new = pl.pallas_call(
        _blend_body,
        grid=(2, B, J),
        in_specs=[
            pl.BlockSpec((1, 1, D, TJ), lambda s, b, j: (s, b, 0, j)),
            pl.BlockSpec((1, 1, D, TJ), lambda s, b, j: (s, b, 0, j)),
            pl.BlockSpec((1, 1, 2, D, TJ), lambda s, b, j: (s, b, 0, 0, j)),
        ],
        out_specs=pl.BlockSpec((1, 1, D, TJ), lambda s, b, j: (s, b, 0, j)),
        out_shape=jax.ShapeDtypeStruct((2, B, D, T), jnp.float32),
    )(xs, avgd, pab)

    atn2 = _attention(new, w1s, b1s, w2s, b2s, w3s, b3s)
    return atn2[0], new[0], atn2[1], new[1]


# speedup vs baseline: 1.0327x; 1.0057x over previous
import jax
import jax.numpy as jnp
from jax.experimental import pallas as pl
from jax.experimental.pallas import tpu as pltpu

_AT = 0.6
_BT = 0.4
_ST = 0.8
_TOPK_RAT = 4
_W = 0.5
_PREC = jax.lax.Precision.DEFAULT
_TJ = 512


def _mm(a, b):
    return jax.lax.dot_general(a, b, (((1,), (0,)), ((), ())),
                               precision=_PREC,
                               preferred_element_type=jnp.float32)


def _mmb(a, b):
    return jax.lax.dot_general(a.astype(jnp.bfloat16), b.astype(jnp.bfloat16),
                               (((1,), (0,)), ((), ())),
                               precision=_PREC,
                               preferred_element_type=jnp.float32)


def _lrelu(x):
    return jnp.where(x >= 0, x, 0.2 * x)


def _shift_right(y, col):
    r = pltpu.roll(y, 1, 1)
    return jnp.where(col >= 1, r, 0.0)


def _shift_left(y, col, T):
    r = pltpu.roll(y, T - 1, 1)
    return jnp.where(col <= T - 2, r, 0.0)


def _attention_body(x_ref, w1_ref, b1_ref, w2_ref, b2_ref, w3_ref, b3_ref,
                    o_ref):
    x = x_ref[0, 0]
    T = x.shape[1]
    w1 = w1_ref[0]
    w2 = w2_ref[0]
    col512 = jax.lax.broadcasted_iota(jnp.int32, (w1.shape[1], T), 1)
    h = _mm(w1[1], x)
    h = h + _shift_right(_mm(w1[0], x), col512)
    h = h + _shift_left(_mm(w1[2], x), col512, T)
    h = _lrelu(h + b1_ref[0])
    g = _mm(w2[1], h)
    g = g + _shift_right(_mm(w2[0], h), col512)
    g = g + _shift_left(_mm(w2[2], h), col512, T)
    g = _lrelu(g + b2_ref[0])
    a = _mm(w3_ref[0], g) + b3_ref[0]
    o_ref[0, 0] = jax.nn.sigmoid(a)


def _adj_body(feat_ref, va_ref, fa_ref, vat_ref, fat_ref,
              aab_ref, amb_ref, avg_ref):
    s = pl.program_id(0)
    j = pl.program_id(2)
    feat = feat_ref[0, 0]
    T = feat.shape[1]
    TJ = aab_ref.shape[-1]
    j0 = j * TJ
    k = T // _TOPK_RAT
    vat = vat_ref[0]
    fat = fat_ref[0]
    va_t = va_ref[0, :, pl.ds(j0, TJ)]
    fa_t = fa_ref[0, :, pl.ds(j0, TJ)]

    armT = ((vat >= _AT) & (fat >= _AT)).astype(jnp.float32)
    brmT = ((vat < _BT) & (fat < _BT)).astype(jnp.float32)
    abrT = armT + brmT
    arm_c = ((va_t >= _AT) & (fa_t >= _AT)).astype(jnp.float32)
    brm_c = ((va_t < _BT) & (fa_t < _BT)).astype(jnp.float32)
    abr_c = arm_c + brm_c
    ambr_c = 1.0 - abr_c

    nrm = jnp.sqrt(jnp.sum(feat * feat, axis=0, keepdims=True))
    fn = feat / jnp.maximum(nrm, 1e-12)
    featc = feat_ref[0, 0, :, pl.ds(j0, TJ)]
    nrm_c = jnp.sqrt(jnp.sum(featc * featc, axis=0, keepdims=True))
    fnc = featc / jnp.maximum(nrm_c, 1e-12)
    S = jax.lax.dot_general(fn, fnc, (((0,), (0,)), ((), ())),
                            precision=_PREC,
                            preferred_element_type=jnp.float32)
    S = jnp.where(S < _ST, 0.0, S)

    def bis(_, carry):
        lo, hi = carry
        mid = 0.5 * (lo + hi)
        cnt = jnp.sum((S >= mid).astype(jnp.float32), axis=0, keepdims=True)
        ge = cnt >= k
        return jnp.where(ge, mid, lo), jnp.where(ge, hi, mid)

    c8 = jnp.sum((S > 0).astype(jnp.float32), axis=0, keepdims=True)
    many = c8 >= k
    lo0 = jnp.where(many, jnp.float32(_ST), 0.0)
    hi0 = jnp.where(many, jnp.float32(1.01), jnp.float32(_ST))
    lo, _ = jax.lax.fori_loop(0, 24, bis, (lo0, hi0))
    S = jnp.where(S >= lo, S, 0.0)

    ii = jax.lax.broadcasted_iota(jnp.int32, (T, TJ), 0)
    jj = jax.lax.broadcasted_iota(jnp.int32, (T, TJ), 1) + j0
    iseye = ii == jj
    eye_m = iseye & (abr_c != 1.0)
    pos = (abrT * ambr_c) > 0
    keep_v = pos | eye_m
    keep_f = (~pos) | eye_m

    def l1n(A):
        cs = jnp.sum(A, axis=0, keepdims=True)
        return A / jnp.maximum(cs, 1e-12)

    A_act = l1n(S * arm_c * armT)
    A_bg = l1n(S * brm_c * brmT)
    A_amb = l1n(jnp.where(s == 0, jnp.where(keep_v, S, 0.0),
                          jnp.where(keep_f, S, 0.0)))

    aab_ref[0, 0, 0] = A_act.astype(jnp.bfloat16)
    aab_ref[0, 0, 1] = A_bg.astype(jnp.bfloat16)
    amb_ref[0, 0] = A_amb.astype(jnp.bfloat16)

    diagv = jnp.sum(jnp.where(iseye, A_amb, 0.0), axis=0, keepdims=True)
    avg = _mmb(feat, A_act + A_bg + A_amb) + featc * diagv
    avg_ref[0, 0] = avg.astype(jnp.bfloat16)


def _gcn_body(feat_ref, w1_ref, b1_ref, w2_ref, b2_ref, aab_ref, amb_ref,
              vat_ref, fat_ref, p_ref):
    feat = feat_ref[0, 0]
    Am = amb_ref[0, 0]
    vat = vat_ref[0]
    fat = fat_ref[0]
    armT = ((vat >= _AT) & (fat >= _AT)).astype(jnp.float32)
    brmT = ((vat < _BT) & (fat < _BT)).astype(jnp.float32)
    ma = jnp.where(jnp.sum(armT) > 0, armT, 0.0)
    mb = jnp.where(jnp.sum(brmT) > 0, brmT, 0.0)
    for r, mask in ((0, ma), (1, mb)):
        A = aab_ref[0, 0, r]
        x = _lrelu(_mmb(_mmb(w1_ref[0, r], feat) + b1_ref[0, r], A))
        x = _lrelu(_mmb(_mmb(w2_ref[0, r], x) + b2_ref[0, r], A))
        y = _mmb(x, Am * mask.astype(jnp.bfloat16))
        p_ref[0, 0, r] = (x + y).astype(jnp.bfloat16)


def _blend_body(feat_ref, avg_ref, p_ref, out_ref):
    feat = feat_ref[0, 0]
    tot = (avg_ref[0, 0].astype(jnp.float32)
           + p_ref[0, 0, 0].astype(jnp.float32)
           + p_ref[0, 0, 1].astype(jnp.float32))
    out_ref[0, 0] = _W * feat + (1.0 - _W) * 0.5 * tot


def _attention(xs, w1s, b1s, w2s, b2s, w3s, b3s):
    S2, B, D, T = xs.shape
    C = w1s.shape[2]
    return pl.pallas_call(
        _attention_body,
        grid=(S2, B),
        in_specs=[
            pl.BlockSpec((1, 1, D, T), lambda s, b: (s, b, 0, 0)),
            pl.BlockSpec((1, 3, C, D), lambda s, b: (s, 0, 0, 0)),
            pl.BlockSpec((1, C, 1), lambda s, b: (s, 0, 0)),
            pl.BlockSpec((1, 3, C, C), lambda s, b: (s, 0, 0, 0)),
            pl.BlockSpec((1, C, 1), lambda s, b: (s, 0, 0)),
            pl.BlockSpec((1, 1, C), lambda s, b: (s, 0, 0)),
            pl.BlockSpec((1, 1, 1), lambda s, b: (s, 0, 0)),
        ],
        out_specs=pl.BlockSpec((1, 1, 1, T), lambda s, b: (s, b, 0, 0)),
        out_shape=jax.ShapeDtypeStruct((S2, B, 1, T), jnp.float32),
    )(xs, w1s, b1s, w2s, b2s, w3s, b3s)


def kernel(vfeat, ffeat, avW1, avb1, avW2, avb2, avW3, avb3,
           afW1, afb1, afW2, afb2, afW3, afb3,
           agvW1, agvb1, agvW2, agvb2, bgvW1, bgvb1, bgvW2, bgvb2,
           agfW1, agfb1, agfW2, agfb2, bgfW1, bgfb1, bgfW2, bgfb2,
           is_training):
    B, D, T = vfeat.shape
    TJ = _TJ
    J = T // TJ
    xs = jnp.stack([vfeat, ffeat])
    w1s = jnp.stack([jnp.transpose(avW1, (2, 0, 1)),
                     jnp.transpose(afW1, (2, 0, 1))])
    b1s = jnp.stack([avb1, afb1])[:, :, None]
    w2s = jnp.stack([jnp.transpose(avW2, (2, 0, 1)),
                     jnp.transpose(afW2, (2, 0, 1))])
    b2s = jnp.stack([avb2, afb2])[:, :, None]
    w3s = jnp.stack([avW3[:, :, 0], afW3[:, :, 0]])
    b3s = jnp.stack([avb3, afb3])[:, :, None]

    atn = _attention(xs, w1s, b1s, w2s, b2s, w3s, b3s)
    vatn, fatn = atn[0], atn[1]
    vatn_t = jnp.transpose(vatn, (0, 2, 1))
    fatn_t = jnp.transpose(fatn, (0, 2, 1))

    aab, aamb, avgd = pl.pallas_call(
        _adj_body,
        grid=(2, B, J),
        in_specs=[
            pl.BlockSpec((1, 1, D, T), lambda s, b, j: (s, b, 0, 0)),
            pl.BlockSpec((1, 1, T), lambda s, b, j: (b, 0, 0)),
            pl.BlockSpec((1, 1, T), lambda s, b, j: (b, 0, 0)),
            pl.BlockSpec((1, T, 1), lambda s, b, j: (b, 0, 0)),
            pl.BlockSpec((1, T, 1), lambda s, b, j: (b, 0, 0)),
        ],
        out_specs=[
            pl.BlockSpec((1, 1, 2, T, TJ), lambda s, b, j: (s, b, 0, 0, j)),
            pl.BlockSpec((1, 1, T, TJ), lambda s, b, j: (s, b, 0, j)),
            pl.BlockSpec((1, 1, D, TJ), lambda s, b, j: (s, b, 0, j)),
        ],
        out_shape=[
            jax.ShapeDtypeStruct((2, B, 2, T, T), jnp.bfloat16),
            jax.ShapeDtypeStruct((2, B, T, T), jnp.bfloat16),
            jax.ShapeDtypeStruct((2, B, D, T), jnp.bfloat16),
        ],
    )(xs, vatn, fatn, vatn_t, fatn_t)

    wg1 = jnp.stack([jnp.stack([agvW1, bgvW1]),
                     jnp.stack([agfW1, bgfW1])]).astype(jnp.bfloat16)
    bg1 = jnp.stack([jnp.stack([agvb1, bgvb1]),
                     jnp.stack([agfb1, bgfb1])])[..., None]
    wg2 = jnp.stack([jnp.stack([agvW2, bgvW2]),
                     jnp.stack([agfW2, bgfW2])]).astype(jnp.bfloat16)
    bg2 = jnp.stack([jnp.stack([agvb2, bgvb2]),
                     jnp.stack([agfb2, bgfb2])])[..., None]

    xs_bf = xs.astype(jnp.bfloat16)
    pab = pl.pallas_call(
        _gcn_body,
        grid=(2, B),
        in_specs=[
            pl.BlockSpec((1, 1, D, T), lambda s, b: (s, b, 0, 0)),
            pl.BlockSpec((1, 2, D, D), lambda s, b: (s, 0, 0, 0)),
            pl.BlockSpec((1, 2, D, 1), lambda s, b: (s, 0, 0, 0)),
            pl.BlockSpec((1, 2, D, D), lambda s, b: (s, 0, 0, 0)),
            pl.BlockSpec((1, 2, D, 1), lambda s, b: (s, 0, 0, 0)),
            pl.BlockSpec((1, 1, 2, T, T), lambda s, b: (s, b, 0, 0, 0)),
            pl.BlockSpec((1, 1, T, T), lambda s, b: (s, b, 0, 0)),
            pl.BlockSpec((1, T, 1), lambda s, b: (b, 0, 0)),
            pl.BlockSpec((1, T, 1), lambda s, b: (b, 0, 0)),
        ],
        out_specs=pl.BlockSpec((1, 1, 2, D, T),
                               lambda s, b: (s, b, 0, 0, 0)),
        out_shape=jax.ShapeDtypeStruct((2, B, 2, D, T), jnp.bfloat16),
    )(xs_bf, wg1, bg1, wg2, bg2, aab, aamb, vatn_t, fatn_t)

    new = pl.pallas_call(
        _blend_body,
        grid=(2, B, J),
        in_specs=[
            pl.BlockSpec((1, 1, D, TJ), lambda s, b, j: (s, b, 0, j)),
            pl.BlockSpec((1, 1, D, TJ), lambda s, b, j: (s, b, 0, j)),
            pl.BlockSpec((1, 1, 2, D, TJ), lambda s, b, j: (s, b, 0, 0, j)),
        ],
        out_specs=pl.BlockSpec((1, 1, D, TJ), lambda s, b, j: (s, b, 0, j)),
        out_shape=jax.ShapeDtypeStruct((2, B, D, T), jnp.float32),
    )(xs, avgd, pab)

    atn2 = _attention(new, w1s, b1s, w2s, b2s, w3s, b3s)
    return atn2[0], new[0], atn2[1], new[1]


# speedup vs baseline: 1.0383x; 1.0055x over previous
import jax
import jax.numpy as jnp
from jax.experimental import pallas as pl
from jax.experimental.pallas import tpu as pltpu

_AT = 0.6
_BT = 0.4
_ST = 0.8
_TOPK_RAT = 4
_W = 0.5
_PREC = jax.lax.Precision.DEFAULT
_TJ = 512


def _mm(a, b):
    return jax.lax.dot_general(a, b, (((1,), (0,)), ((), ())),
                               precision=_PREC,
                               preferred_element_type=jnp.float32)


def _mmb(a, b):
    return jax.lax.dot_general(a.astype(jnp.bfloat16), b.astype(jnp.bfloat16),
                               (((1,), (0,)), ((), ())),
                               precision=_PREC,
                               preferred_element_type=jnp.float32)


def _lrelu(x):
    return jnp.where(x >= 0, x, 0.2 * x)


def _shift_right(y, col):
    r = pltpu.roll(y, 1, 1)
    return jnp.where(col >= 1, r, 0.0)


def _shift_left(y, col, T):
    r = pltpu.roll(y, T - 1, 1)
    return jnp.where(col <= T - 2, r, 0.0)


def _attention_body(x_ref, w1_ref, b1_ref, w2_ref, b2_ref, w3_ref, b3_ref,
                    o_ref):
    x = x_ref[0, 0]
    T = x.shape[1]
    w1 = w1_ref[0]
    w2 = w2_ref[0]
    col512 = jax.lax.broadcasted_iota(jnp.int32, (w1.shape[1], T), 1)
    h = _mm(w1[1], x)
    h = h + _shift_right(_mm(w1[0], x), col512)
    h = h + _shift_left(_mm(w1[2], x), col512, T)
    h = _lrelu(h + b1_ref[0])
    g = _mm(w2[1], h)
    g = g + _shift_right(_mm(w2[0], h), col512)
    g = g + _shift_left(_mm(w2[2], h), col512, T)
    g = _lrelu(g + b2_ref[0])
    a = _mm(w3_ref[0], g) + b3_ref[0]
    o_ref[0, 0] = jax.nn.sigmoid(a)


def _adj_body(feat_ref, va_ref, fa_ref, vat_ref, fat_ref,
              aab_ref, amb_ref, avg_ref):
    s = pl.program_id(0)
    j = pl.program_id(2)
    feat = feat_ref[0, 0]
    T = feat.shape[1]
    TJ = aab_ref.shape[-1]
    j0 = j * TJ
    k = T // _TOPK_RAT
    vat = vat_ref[0]
    fat = fat_ref[0]
    va_t = va_ref[0, :, pl.ds(j0, TJ)]
    fa_t = fa_ref[0, :, pl.ds(j0, TJ)]

    armT = ((vat >= _AT) & (fat >= _AT)).astype(jnp.float32)
    brmT = ((vat < _BT) & (fat < _BT)).astype(jnp.float32)
    abrT = armT + brmT
    arm_c = ((va_t >= _AT) & (fa_t >= _AT)).astype(jnp.float32)
    brm_c = ((va_t < _BT) & (fa_t < _BT)).astype(jnp.float32)
    abr_c = arm_c + brm_c
    ambr_c = 1.0 - abr_c

    nrm = jnp.sqrt(jnp.sum(feat * feat, axis=0, keepdims=True))
    fn = feat / jnp.maximum(nrm, 1e-12)
    featc = feat_ref[0, 0, :, pl.ds(j0, TJ)]
    nrm_c = jnp.sqrt(jnp.sum(featc * featc, axis=0, keepdims=True))
    fnc = featc / jnp.maximum(nrm_c, 1e-12)
    S = jax.lax.dot_general(fn, fnc, (((0,), (0,)), ((), ())),
                            precision=_PREC,
                            preferred_element_type=jnp.float32)
    S = jnp.where(S < _ST, 0.0, S)

    def bis(_, carry):
        lo, hi = carry
        mid = 0.5 * (lo + hi)
        cnt = jnp.sum((S >= mid).astype(jnp.float32), axis=0, keepdims=True)
        ge = cnt >= k
        return jnp.where(ge, mid, lo), jnp.where(ge, hi, mid)

    c8 = jnp.sum((S > 0).astype(jnp.float32), axis=0, keepdims=True)
    many = c8 >= k
    lo0 = jnp.where(many, jnp.float32(_ST), 0.0)
    hi0 = jnp.where(many, jnp.float32(1.01), jnp.float32(_ST))
    lo, _ = jax.lax.fori_loop(0, 24, bis, (lo0, hi0))
    S = jnp.where(S >= lo, S, 0.0)

    ii = jax.lax.broadcasted_iota(jnp.int32, (T, TJ), 0)
    jj = jax.lax.broadcasted_iota(jnp.int32, (T, TJ), 1) + j0
    iseye = ii == jj
    eye_m = iseye & (abr_c != 1.0)
    pos = (abrT * ambr_c) > 0
    keep_v = pos | eye_m
    keep_f = (~pos) | eye_m

    def l1n(A):
        cs = jnp.sum(A, axis=0, keepdims=True)
        return A / jnp.maximum(cs, 1e-12)

    A_act = l1n(S * arm_c * armT)
    A_bg = l1n(S * brm_c * brmT)
    A_amb = l1n(jnp.where(s == 0, jnp.where(keep_v, S, 0.0),
                          jnp.where(keep_f, S, 0.0)))

    aab_ref[0, 0, 0] = A_act.astype(jnp.bfloat16)
    aab_ref[0, 0, 1] = A_bg.astype(jnp.bfloat16)
    amb_ref[0, 0] = A_amb.astype(jnp.bfloat16)

    diagv = jnp.sum(jnp.where(iseye, A_amb, 0.0), axis=0, keepdims=True)
    avg = _mmb(feat, A_act + A_bg + A_amb) + featc * diagv
    avg_ref[0, 0] = avg.astype(jnp.bfloat16)


def _gcn_body(feat_ref, w1_ref, b1_ref, w2_ref, b2_ref, a_ref, amb_ref,
              vat_ref, fat_ref, p_ref):
    r = pl.program_id(1)
    feat = feat_ref[0, 0]
    A = a_ref[0, 0, 0]
    x = _lrelu(_mmb(_mmb(w1_ref[0, 0], feat) + b1_ref[0, 0], A))
    x = _lrelu(_mmb(_mmb(w2_ref[0, 0], x) + b2_ref[0, 0], A))
    Am = amb_ref[0, 0]
    vat = vat_ref[0]
    fat = fat_ref[0]
    armT = ((vat >= _AT) & (fat >= _AT)).astype(jnp.float32)
    brmT = ((vat < _BT) & (fat < _BT)).astype(jnp.float32)
    ma = jnp.where(jnp.sum(armT) > 0, armT, 0.0)
    mb = jnp.where(jnp.sum(brmT) > 0, brmT, 0.0)
    mask = jnp.where(r == 0, ma, mb).astype(jnp.bfloat16)
    y = _mmb(x, Am * mask)
    p_ref[0, 0, 0] = (x + y).astype(jnp.bfloat16)


def _blend_body(feat_ref, avg_ref, p_ref, out_ref):
    feat = feat_ref[0, 0]
    tot = (avg_ref[0, 0].astype(jnp.float32)
           + p_ref[0, 0, 0].astype(jnp.float32)
           + p_ref[0, 0, 1].astype(jnp.float32))
    out_ref[0, 0] = _W * feat + (1.0 - _W) * 0.5 * tot


def _attention(xs, w1s, b1s, w2s, b2s, w3s, b3s):
    S2, B, D, T = xs.shape
    C = w1s.shape[2]
    return pl.pallas_call(
        _attention_body,
        grid=(S2, B),
        in_specs=[
            pl.BlockSpec((1, 1, D, T), lambda s, b: (s, b, 0, 0)),
            pl.BlockSpec((1, 3, C, D), lambda s, b: (s, 0, 0, 0)),
            pl.BlockSpec((1, C, 1), lambda s, b: (s, 0, 0)),
            pl.BlockSpec((1, 3, C, C), lambda s, b: (s, 0, 0, 0)),
            pl.BlockSpec((1, C, 1), lambda s, b: (s, 0, 0)),
            pl.BlockSpec((1, 1, C), lambda s, b: (s, 0, 0)),
            pl.BlockSpec((1, 1, 1), lambda s, b: (s, 0, 0)),
        ],
        out_specs=pl.BlockSpec((1, 1, 1, T), lambda s, b: (s, b, 0, 0)),
        out_shape=jax.ShapeDtypeStruct((S2, B, 1, T), jnp.float32),
    )(xs, w1s, b1s, w2s, b2s, w3s, b3s)


def kernel(vfeat, ffeat, avW1, avb1, avW2, avb2, avW3, avb3,
           afW1, afb1, afW2, afb2, afW3, afb3,
           agvW1, agvb1, agvW2, agvb2, bgvW1, bgvb1, bgvW2, bgvb2,
           agfW1, agfb1, agfW2, agfb2, bgfW1, bgfb1, bgfW2, bgfb2,
           is_training):
    B, D, T = vfeat.shape
    TJ = _TJ
    J = T // TJ
    xs = jnp.stack([vfeat, ffeat])
    w1s = jnp.stack([jnp.transpose(avW1, (2, 0, 1)),
                     jnp.transpose(afW1, (2, 0, 1))])
    b1s = jnp.stack([avb1, afb1])[:, :, None]
    w2s = jnp.stack([jnp.transpose(avW2, (2, 0, 1)),
                     jnp.transpose(afW2, (2, 0, 1))])
    b2s = jnp.stack([avb2, afb2])[:, :, None]
    w3s = jnp.stack([avW3[:, :, 0], afW3[:, :, 0]])
    b3s = jnp.stack([avb3, afb3])[:, :, None]

    atn = _attention(xs, w1s, b1s, w2s, b2s, w3s, b3s)
    vatn, fatn = atn[0], atn[1]
    vatn_t = jnp.transpose(vatn, (0, 2, 1))
    fatn_t = jnp.transpose(fatn, (0, 2, 1))

    aab, aamb, avgd = pl.pallas_call(
        _adj_body,
        grid=(2, B, J),
        in_specs=[
            pl.BlockSpec((1, 1, D, T), lambda s, b, j: (s, b, 0, 0)),
            pl.BlockSpec((1, 1, T), lambda s, b, j: (b, 0, 0)),
            pl.BlockSpec((1, 1, T), lambda s, b, j: (b, 0, 0)),
            pl.BlockSpec((1, T, 1), lambda s, b, j: (b, 0, 0)),
            pl.BlockSpec((1, T, 1), lambda s, b, j: (b, 0, 0)),
        ],
        out_specs=[
            pl.BlockSpec((1, 1, 2, T, TJ), lambda s, b, j: (s, b, 0, 0, j)),
            pl.BlockSpec((1, 1, T, TJ), lambda s, b, j: (s, b, 0, j)),
            pl.BlockSpec((1, 1, D, TJ), lambda s, b, j: (s, b, 0, j)),
        ],
        out_shape=[
            jax.ShapeDtypeStruct((2, B, 2, T, T), jnp.bfloat16),
            jax.ShapeDtypeStruct((2, B, T, T), jnp.bfloat16),
            jax.ShapeDtypeStruct((2, B, D, T), jnp.bfloat16),
        ],
    )(xs, vatn, fatn, vatn_t, fatn_t)

    wg1 = jnp.stack([jnp.stack([agvW1, bgvW1]),
                     jnp.stack([agfW1, bgfW1])]).astype(jnp.bfloat16)
    bg1 = jnp.stack([jnp.stack([agvb1, bgvb1]),
                     jnp.stack([agfb1, bgfb1])])[..., None]
    wg2 = jnp.stack([jnp.stack([agvW2, bgvW2]),
                     jnp.stack([agfW2, bgfW2])]).astype(jnp.bfloat16)
    bg2 = jnp.stack([jnp.stack([agvb2, bgvb2]),
                     jnp.stack([agfb2, bgfb2])])[..., None]

    xs_bf = xs.astype(jnp.bfloat16)
    pab = pl.pallas_call(
        _gcn_body,
        grid=(2, 2, B),
        in_specs=[
            pl.BlockSpec((1, 1, D, T), lambda s, r, b: (s, b, 0, 0)),
            pl.BlockSpec((1, 1, D, D), lambda s, r, b: (s, r, 0, 0)),
            pl.BlockSpec((1, 1, D, 1), lambda s, r, b: (s, r, 0, 0)),
            pl.BlockSpec((1, 1, D, D), lambda s, r, b: (s, r, 0, 0)),
            pl.BlockSpec((1, 1, D, 1), lambda s, r, b: (s, r, 0, 0)),
            pl.BlockSpec((1, 1, 1, T, T), lambda s, r, b: (s, b, r, 0, 0)),
            pl.BlockSpec((1, 1, T, T), lambda s, r, b: (s, b, 0, 0)),
            pl.BlockSpec((1, T, 1), lambda s, r, b: (b, 0, 0)),
            pl.BlockSpec((1, T, 1), lambda s, r, b: (b, 0, 0)),
        ],
        out_specs=pl.BlockSpec((1, 1, 1, D, T),
                               lambda s, r, b: (s, b, r, 0, 0)),
        out_shape=jax.ShapeDtypeStruct((2, B, 2, D, T), jnp.bfloat16),
    )(xs_bf, wg1, bg1, wg2, bg2, aab, aamb, vatn_t, fatn_t)

    new = pl.pallas_call(
        _blend_body,
        grid=(2, B, J),
        in_specs=[
            pl.BlockSpec((1, 1, D, TJ), lambda s, b, j: (s, b, 0, j)),
            pl.BlockSpec((1, 1, D, TJ), lambda s, b, j: (s, b, 0, j)),
            pl.BlockSpec((1, 1, 2, D, TJ), lambda s, b, j: (s, b, 0, 0, j)),
        ],
        out_specs=pl.BlockSpec((1, 1, D, TJ), lambda s, b, j: (s, b, 0, j)),
        out_shape=jax.ShapeDtypeStruct((2, B, D, T), jnp.float32),
    )(xs, avgd, pab)

    atn2 = _attention(new, w1s, b1s, w2s, b2s, w3s, b3s)
    return atn2[0], new[0], atn2[1], new[1]
